# baseline (speedup 1.0000x reference)
import functools

import jax
import jax.numpy as jnp
from jax import lax
from jax.experimental import pallas as pl
from jax.experimental.pallas import tpu as pltpu
from jax.experimental.pallas import tpu_sc as plsc

_CHUNK = 32
_NBUF = 2
_SBLK = 512


def kernel(token_ids, weight):
    batch_size, seq_len = token_ids.shape
    emb_dim = weight.shape[1]
    n_tc = batch_size - 1

    info = plsc.get_sparse_core_info()
    num_workers = info.num_cores * info.num_subcores
    rows_per = seq_len // num_workers
    nch = rows_per // _CHUNK

    mesh = plsc.VectorSubcoreMesh(core_axis_name="c", subcore_axis_name="s")

    @functools.partial(
        pl.kernel,
        mesh=mesh,
        out_type=jax.ShapeDtypeStruct((seq_len, emb_dim), weight.dtype),
        scratch_types=[
            pltpu.VMEM((_NBUF, _CHUNK, emb_dim), weight.dtype),
            pltpu.SemaphoreType.DMA,
            pltpu.SemaphoreType.DMA,
        ],
    )
    def sc_copy(w_hbm, out_hbm, buf, gsem, ssem):
        wid = lax.axis_index("s") * info.num_cores + lax.axis_index("c")
        base = wid * rows_per

        gh = [None] * nch
        for i in range(_NBUF):
            gh[i] = pltpu.async_copy(
                w_hbm.at[pl.ds(base + i * _CHUNK, _CHUNK)], buf.at[i], gsem
            )
        for i in range(nch):
            gh[i].wait()
            s = pltpu.async_copy(
                buf.at[i % _NBUF],
                out_hbm.at[pl.ds(base + i * _CHUNK, _CHUNK)],
                ssem,
            )
            s.wait()
            nxt = i + _NBUF
            if nxt < nch:
                gh[nxt] = pltpu.async_copy(
                    w_hbm.at[pl.ds(base + nxt * _CHUNK, _CHUNK)],
                    buf.at[nxt % _NBUF],
                    gsem,
                )

    sc_out = sc_copy(weight)

    def tc_body(w_ref, o_ref):
        o_ref[...] = jnp.broadcast_to(w_ref[...][None], o_ref.shape)

    out_partial = pl.pallas_call(
        tc_body,
        grid=(seq_len // _SBLK,),
        in_specs=[pl.BlockSpec((_SBLK, emb_dim), lambda i: (i, 0))],
        out_specs=pl.BlockSpec((n_tc, _SBLK, emb_dim), lambda i: (0, i, 0)),
        out_shape=jax.ShapeDtypeStruct((batch_size, seq_len, emb_dim), weight.dtype),
    )(weight)

    def fill_body(prev_ref, sc_ref, o_ref):
        del prev_ref
        o_ref[...] = sc_ref[...][None]

    return pl.pallas_call(
        fill_body,
        grid=(seq_len // _SBLK,),
        in_specs=[
            pl.BlockSpec(memory_space=pl.ANY),
            pl.BlockSpec((_SBLK, emb_dim), lambda i: (i, 0)),
        ],
        out_specs=pl.BlockSpec(
            (1, _SBLK, emb_dim), lambda i: (n_tc, i, 0)
        ),
        out_shape=jax.ShapeDtypeStruct((batch_size, seq_len, emb_dim), weight.dtype),
        input_output_aliases={0: 0},
    )(out_partial, sc_out)


# speedup vs baseline: 1.4251x; 1.4251x over previous
import functools

import jax
import jax.numpy as jnp
from jax import lax
from jax.experimental import pallas as pl
from jax.experimental.pallas import tpu as pltpu
from jax.experimental.pallas import tpu_sc as plsc

_CHUNK = 32
_NBUF = 3


def kernel(token_ids, weight):
    batch_size, seq_len = token_ids.shape
    emb_dim = weight.shape[1]

    info = plsc.get_sparse_core_info()
    num_workers = info.num_cores * info.num_subcores
    rows_per = seq_len // num_workers
    nch = rows_per // _CHUNK

    mesh = plsc.VectorSubcoreMesh(core_axis_name="c", subcore_axis_name="s")

    @functools.partial(
        pl.kernel,
        mesh=mesh,
        out_type=jax.ShapeDtypeStruct((batch_size, seq_len, emb_dim), weight.dtype),
        scratch_types=[
            pltpu.VMEM((_NBUF, _CHUNK, emb_dim), weight.dtype),
            pltpu.SemaphoreType.DMA,
            pltpu.SemaphoreType.DMA,
        ],
    )
    def copy_kernel(w_hbm, out_hbm, buf, gsem, ssem):
        wid = lax.axis_index("s") * info.num_cores + lax.axis_index("c")
        base = wid * rows_per

        def gather(i):
            return pltpu.async_copy(
                w_hbm.at[pl.ds(base + i * _CHUNK, _CHUNK)],
                buf.at[i % _NBUF],
                gsem,
            )

        gh = [None] * nch
        sh = [None] * nch
        for i in range(2):
            gh[i] = gather(i)
        for i in range(nch):
            gh[i].wait()
            sh[i] = [
                pltpu.async_copy(
                    buf.at[i % _NBUF],
                    out_hbm.at[b, pl.ds(base + i * _CHUNK, _CHUNK)],
                    ssem,
                )
                for b in range(batch_size)
            ]
            nxt = i + 2
            if nxt < nch:
                prev = nxt - _NBUF
                if prev >= 0:
                    for s in sh[prev]:
                        s.wait()
                gh[nxt] = gather(nxt)
        for i in range(max(nch - _NBUF, 0), nch):
            for s in sh[i]:
                s.wait()

    return copy_kernel(weight)


# speedup vs baseline: 1.4325x; 1.0051x over previous
import functools

import jax
import jax.numpy as jnp
from jax import lax
from jax.experimental import pallas as pl
from jax.experimental.pallas import tpu as pltpu
from jax.experimental.pallas import tpu_sc as plsc

_CHUNK = 32
_NBUF = 3


def kernel(token_ids, weight):
    batch_size, seq_len = token_ids.shape
    emb_dim = weight.shape[1]

    info = plsc.get_sparse_core_info()
    num_workers = info.num_cores * info.num_subcores
    rows_per = seq_len // num_workers
    nch = rows_per // _CHUNK

    mesh = plsc.VectorSubcoreMesh(core_axis_name="c", subcore_axis_name="s")

    @functools.partial(
        pl.kernel,
        mesh=mesh,
        out_type=jax.ShapeDtypeStruct((batch_size, seq_len, emb_dim), weight.dtype),
        scratch_types=[
            pltpu.VMEM((_NBUF, _CHUNK, emb_dim), weight.dtype),
            pltpu.SemaphoreType.DMA,
            pltpu.SemaphoreType.DMA,
        ],
    )
    def copy_kernel(w_hbm, out_hbm, buf, gsem, ssem):
        wid = lax.axis_index("s") * info.num_cores + lax.axis_index("c")
        base = wid * rows_per

        def gather(i):
            return pltpu.async_copy(
                w_hbm.at[pl.ds(base + i * _CHUNK, _CHUNK)],
                buf.at[i % _NBUF],
                gsem,
            )

        gh = [None] * nch
        sh = [None] * nch
        for i in range(min(2, nch)):
            gh[i] = gather(i)
        for i in range(nch):
            gh[i].wait()
            sh[i] = [
                pltpu.async_copy(
                    buf.at[i % _NBUF],
                    out_hbm.at[b, pl.ds(base + i * _CHUNK, _CHUNK)],
                    ssem,
                )
                for b in range(batch_size)
            ]
            nxt = i + 2
            if nxt < nch:
                prev = nxt - _NBUF
                if prev >= 0:
                    for s in sh[prev]:
                        s.wait()
                gh[nxt] = gather(nxt)
        for i in range(max(nch - _NBUF, 0), nch):
            for s in sh[i]:
                s.wait()

    return copy_kernel(weight)
